# baseline (speedup 1.0000x reference)
import jax
import jax.numpy as jnp
from jax import lax
from jax.experimental import pallas as pl
from jax.experimental.pallas import tpu as pltpu
from jax.experimental.pallas import tpu_sc as plsc

N = 10000
E = 160000
D = 256
HID = 256
NCLS = 32
NEG = 0.2

NTILES = 16
EPT = E // NTILES
BCH = 80
NCH = EPT // BCH
NROW = 80
NRCH = N // NROW
NRPT = -(-NRCH // NTILES)

f32 = jnp.float32
i32 = jnp.int32

BN = 2000
TCGRID = N // BN


def _tc1_body(x_ref, w0_ref, ssa_ref, sda_ref, ssb_ref, sdb_ref,
              ha_ref, hb_ref, asa_ref, ada_ref, asb_ref, adb_ref):
    h = jnp.dot(x_ref[...], w0_ref[...], preferred_element_type=f32)
    ha_ref[...] = h[:, :128]
    hb_ref[...] = h[:, 128:]
    asa_ref[...] = jnp.dot(h, ssa_ref[...], preferred_element_type=f32)
    ada_ref[...] = jnp.dot(h, sda_ref[...], preferred_element_type=f32)
    asb_ref[...] = jnp.dot(h, ssb_ref[...], preferred_element_type=f32)
    adb_ref[...] = jnp.dot(h, sdb_ref[...], preferred_element_type=f32)


def _tc1(x, w0, ssa, sda, ssb, sdb):
    full = lambda s: pl.BlockSpec(s, lambda i: (0, 0))
    row = lambda w: pl.BlockSpec((BN, w), lambda i: (i, 0))
    return pl.pallas_call(
        _tc1_body,
        grid=(TCGRID,),
        in_specs=[row(D), full((D, HID)), full((HID, 16)), full((HID, 16)),
                  full((HID, 16)), full((HID, 16))],
        out_specs=[row(128), row(128), row(16), row(16), row(16), row(16)],
        out_shape=[jax.ShapeDtypeStruct((N, 128), f32),
                   jax.ShapeDtypeStruct((N, 128), f32),
                   jax.ShapeDtypeStruct((N, 16), f32),
                   jax.ShapeDtypeStruct((N, 16), f32),
                   jax.ShapeDtypeStruct((N, 16), f32),
                   jax.ShapeDtypeStruct((N, 16), f32)],
    )(x, w0, ssa, sda, ssb, sdb)


def _tc2_body(ma_ref, mb_ref, da_ref, db_ref, b0a_ref, b0b_ref,
              w1a_ref, w1b_ref, ps_ref, pd_ref,
              h1a_ref, h1b_ref, s1_ref, d1_ref):
    eps = 1e-16

    def norm(m_ref, d_ref, b_ref):
        d = d_ref[...]
        den = jnp.concatenate(
            [jnp.broadcast_to(d[:, 0:1] + eps, (BN, 64)),
             jnp.broadcast_to(d[:, 1:2] + eps, (BN, 64))], axis=1)
        v = m_ref[...] / den + b_ref[...]
        return jnp.where(v > 0, v, jnp.exp(v) - 1.0)

    h0a = norm(ma_ref, da_ref, b0a_ref)
    h0b = norm(mb_ref, db_ref, b0b_ref)
    h1 = (jnp.dot(h0a, w1a_ref[...], preferred_element_type=f32) +
          jnp.dot(h0b, w1b_ref[...], preferred_element_type=f32))
    h1a_ref[...] = h1[:, :16]
    h1b_ref[...] = h1[:, 16:]
    s1_ref[...] = jnp.dot(h1, ps_ref[...], preferred_element_type=f32)
    d1_ref[...] = jnp.dot(h1, pd_ref[...], preferred_element_type=f32)


def _tc2(ma, mb, da, db, b0a, b0b, w1a, w1b, ps, pd):
    full = lambda s: pl.BlockSpec(s, lambda i: (0, 0))
    row = lambda w: pl.BlockSpec((BN, w), lambda i: (i, 0))
    return pl.pallas_call(
        _tc2_body,
        grid=(TCGRID,),
        in_specs=[row(128), row(128), row(16), row(16),
                  full((1, 128)), full((1, 128)),
                  full((128, NCLS)), full((128, NCLS)),
                  full((NCLS, 16)), full((NCLS, 16))],
        out_specs=[row(16), row(16), row(16), row(16)],
        out_shape=[jax.ShapeDtypeStruct((N, 16), f32),
                   jax.ShapeDtypeStruct((N, 16), f32),
                   jax.ShapeDtypeStruct((N, 16), f32),
                   jax.ShapeDtypeStruct((N, 16), f32)],
    )(ma, mb, da, db, b0a, b0b, w1a, w1b, ps, pd)


_GDN = lax.GatherDimensionNumbers(
    offset_dims=(), collapsed_slice_dims=(0,), start_index_map=(0,))


def _splat(v, j):
    idx = jnp.full((16, 1), j, dtype=i32)
    return lax.gather(v, idx, _GDN, (1,),
                      mode=lax.GatherScatterMode.PROMISE_IN_BOUNDS)


def _mesh():
    return plsc.VectorSubcoreMesh(core_axis_name="c", subcore_axis_name="s",
                                  num_cores=2, num_subcores=16)


def _cp():
    return pltpu.CompilerParams(use_tc_tiling_on_sc=False,
                                needs_layout_passes=False)


def _sc0_body(ha, hb, asa, ada, asb, adb, srch, dsth,
              msga, msgb, dena, denb,
              sidx2, didx2, av, dv, exv, rows, msg_sh, den_sh, gsem, ssem):
    c = lax.axis_index("c")
    s = lax.axis_index("s")
    lanelt2 = lax.broadcasted_iota(i32, (16,), 0) < 2
    zero16 = jnp.zeros((16,), f32)

    def zb(i, _):
        for k in range(8):
            rows[i, pl.ds(k * 16, 16)] = zero16
        exv[i, :] = zero16
        return 0
    lax.fori_loop(0, BCH, zb, 0)

    for k in range(NRPT):
        cid = s + NTILES * k

        def zc(lo=pl.multiple_of(cid * NROW, 8)):
            pltpu.sync_copy(rows, msg_sh.at[pl.ds(lo, NROW)])
            pltpu.sync_copy(exv, den_sh.at[pl.ds(lo, NROW)])
        pl.when(cid < NRCH)(zc)

    pltpu.sync_copy(srch.at[s], sidx2)
    pltpu.sync_copy(dsth.at[s], didx2)
    plsc.subcore_barrier()

    def run(h_t, as_t, ad_t):
        def chunk(j, _):
            si = sidx2.at[j]
            di = didx2.at[j]
            g1 = pltpu.async_copy(as_t.at[si], av, gsem)
            g2 = pltpu.async_copy(ad_t.at[di], dv, gsem)
            g3 = pltpu.async_copy(h_t.at[si], rows, gsem)
            g1.wait()
            g2.wait()
            g3.wait()

            def edge(i, _):
                e = av[i, :] + dv[i, :]
                e = jnp.where(e > 0, e, NEG * e)
                ex = jnp.where(lanelt2, jnp.exp(e), 0.0)
                exv[i, :] = ex
                s0 = _splat(ex, 0)
                s1 = _splat(ex, 1)
                for k in range(4):
                    rows[i, pl.ds(k * 16, 16)] = rows[i, pl.ds(k * 16, 16)] * s0
                for k in range(4, 8):
                    rows[i, pl.ds(k * 16, 16)] = rows[i, pl.ds(k * 16, 16)] * s1
                return 0
            lax.fori_loop(0, BCH, edge, 0, unroll=4)
            s1_ = pltpu.async_copy(exv, den_sh.at[di], ssem, add=True)
            s2_ = pltpu.async_copy(rows, msg_sh.at[di], ssem, add=True)
            s1_.wait()
            s2_.wait()
            return 0
        lax.fori_loop(0, NCH, chunk, 0)

    pl.when(c == 0)(lambda: run(ha, asa, ada))
    pl.when(c == 1)(lambda: run(hb, asb, adb))
    plsc.subcore_barrier()

    def writeout(msg_o, den_o):
        for k in range(NRPT):
            cid = s + NTILES * k

            def wc(lo=pl.multiple_of((s + NTILES * k) * NROW, 8)):
                pltpu.sync_copy(msg_sh.at[pl.ds(lo, NROW)], rows)
                pltpu.sync_copy(rows, msg_o.at[pl.ds(lo, NROW)])
                pltpu.sync_copy(den_sh.at[pl.ds(lo, NROW)], exv)
                pltpu.sync_copy(exv, den_o.at[pl.ds(lo, NROW)])
            pl.when(cid < NRCH)(wc)

    pl.when(c == 0)(lambda: writeout(msga, dena))
    pl.when(c == 1)(lambda: writeout(msgb, denb))


def _sc0(ha, hb, asa, ada, asb, adb, src3, dst3):
    fn = pl.kernel(
        _sc0_body,
        out_type=[jax.ShapeDtypeStruct((N, 128), f32),
                  jax.ShapeDtypeStruct((N, 128), f32),
                  jax.ShapeDtypeStruct((N, 16), f32),
                  jax.ShapeDtypeStruct((N, 16), f32)],
        mesh=_mesh(),
        compiler_params=_cp(),
        scratch_types=[pltpu.VMEM((NCH, BCH), i32),
                       pltpu.VMEM((NCH, BCH), i32),
                       pltpu.VMEM((BCH, 16), f32),
                       pltpu.VMEM((BCH, 16), f32),
                       pltpu.VMEM((BCH, 16), f32),
                       pltpu.VMEM((BCH, 128), f32),
                       pltpu.VMEM_SHARED((N, 128), f32),
                       pltpu.VMEM_SHARED((N, 16), f32),
                       pltpu.SemaphoreType.DMA,
                       pltpu.SemaphoreType.DMA],
    )
    return fn(ha, hb, asa, ada, asb, adb, src3, dst3)


def _sc1_body(h1a, h1b, s1t, d1t, srch, dsth, b1h,
              o1a, o1b,
              sidx2, didx2, av, dv, exv, rows, accb, denb_v, bvec,
              acc_sh, den_sh, gsem, ssem):
    c = lax.axis_index("c")
    s = lax.axis_index("s")
    lane0 = lax.broadcasted_iota(i32, (16,), 0) < 1
    zero16 = jnp.zeros((16,), f32)

    def zb(i, _):
        rows[i, :] = zero16
        exv[i, :] = zero16
        return 0
    lax.fori_loop(0, BCH, zb, 0)

    for k in range(NRPT):
        cid = s + NTILES * k

        def zc(lo=pl.multiple_of(cid * NROW, 8)):
            pltpu.sync_copy(rows, acc_sh.at[pl.ds(lo, NROW)])
            pltpu.sync_copy(exv, den_sh.at[pl.ds(lo, NROW)])
        pl.when(cid < NRCH)(zc)

    pltpu.sync_copy(srch.at[s], sidx2)
    pltpu.sync_copy(dsth.at[s], didx2)
    plsc.subcore_barrier()

    def run(h_t):
        def chunk(j, _):
            si = sidx2.at[j]
            di = didx2.at[j]
            g1 = pltpu.async_copy(s1t.at[si], av, gsem)
            g2 = pltpu.async_copy(d1t.at[di], dv, gsem)
            g3 = pltpu.async_copy(h_t.at[si], rows, gsem)
            g1.wait()
            g2.wait()
            g3.wait()

            def edge(i, _):
                e = av[i, :] + dv[i, :]
                e = jnp.where(e > 0, e, NEG * e)
                ex = jnp.where(lane0, jnp.exp(e), 0.0)
                exv[i, :] = ex
                rows[i, :] = rows[i, :] * _splat(ex, 0)
                return 0
            lax.fori_loop(0, BCH, edge, 0, unroll=4)
            s1_ = pltpu.async_copy(exv, den_sh.at[di], ssem, add=True)
            s2_ = pltpu.async_copy(rows, acc_sh.at[di], ssem, add=True)
            s1_.wait()
            s2_.wait()
            return 0
        lax.fori_loop(0, NCH, chunk, 0)

    pl.when(c == 0)(lambda: run(h1a))
    pl.when(c == 1)(lambda: run(h1b))
    plsc.subcore_barrier()

    pltpu.sync_copy(b1h.at[pl.ds(pl.multiple_of(c * 16, 16), 16)], bvec)
    bv = bvec[:]

    def writeout(o_ref):
        for k in range(NRPT):
            cid = s + NTILES * k

            def wc(lo=pl.multiple_of((s + NTILES * k) * NROW, 8)):
                pltpu.sync_copy(acc_sh.at[pl.ds(lo, NROW)], accb)
                pltpu.sync_copy(den_sh.at[pl.ds(lo, NROW)], denb_v)

                def node(i, _):
                    dn = _splat(denb_v[i, :], 0) + 1e-16
                    accb[i, :] = accb[i, :] / dn + bv
                    return 0
                lax.fori_loop(0, NROW, node, 0)
                pltpu.sync_copy(accb, o_ref.at[pl.ds(lo, NROW)])
            pl.when(cid < NRCH)(wc)

    pl.when(c == 0)(lambda: writeout(o1a))
    pl.when(c == 1)(lambda: writeout(o1b))


def _sc1(h1a, h1b, s1t, d1t, src3, dst3, b1):
    fn = pl.kernel(
        _sc1_body,
        out_type=[jax.ShapeDtypeStruct((N, 16), f32),
                  jax.ShapeDtypeStruct((N, 16), f32)],
        mesh=_mesh(),
        compiler_params=_cp(),
        scratch_types=[pltpu.VMEM((NCH, BCH), i32),
                       pltpu.VMEM((NCH, BCH), i32),
                       pltpu.VMEM((BCH, 16), f32),
                       pltpu.VMEM((BCH, 16), f32),
                       pltpu.VMEM((BCH, 16), f32),
                       pltpu.VMEM((BCH, 16), f32),
                       pltpu.VMEM((NROW, 16), f32),
                       pltpu.VMEM((NROW, 16), f32),
                       pltpu.VMEM((16,), f32),
                       pltpu.VMEM_SHARED((N, 16), f32),
                       pltpu.VMEM_SHARED((N, 16), f32),
                       pltpu.SemaphoreType.DMA,
                       pltpu.SemaphoreType.DMA],
    )
    return fn(h1a, h1b, s1t, d1t, src3, dst3, b1)


def _head_sel(att, heads):
    sel = jnp.zeros((HID, 16), f32)
    for j, h in enumerate(heads):
        sel = sel.at[h * 64:(h + 1) * 64, j].set(att[h])
    return sel


def kernel(x, edge_index, W0, att_src0, att_dst0, b0, W1, att_src1,
           att_dst1, b1):
    src3 = edge_index[0].astype(i32).reshape(NTILES, NCH, BCH)
    dst3 = edge_index[1].astype(i32).reshape(NTILES, NCH, BCH)

    ssa = _head_sel(att_src0, (0, 1))
    sda = _head_sel(att_dst0, (0, 1))
    ssb = _head_sel(att_src0, (2, 3))
    sdb = _head_sel(att_dst0, (2, 3))

    ha, hb, asa, ada, asb, adb = _tc1(x, W0, ssa, sda, ssb, sdb)
    msga, msgb, dena, denb = _sc0(ha, hb, asa, ada, asb, adb, src3, dst3)

    b0a = b0[:128].reshape(1, 128)
    b0b = b0[128:].reshape(1, 128)
    w1a = W1[:128]
    w1b = W1[128:]
    ps = jnp.zeros((NCLS, 16), f32).at[:, 0].set(att_src1[0])
    pd = jnp.zeros((NCLS, 16), f32).at[:, 0].set(att_dst1[0])

    h1a, h1b, s1t, d1t = _tc2(msga, msgb, dena, denb, b0a, b0b, w1a, w1b,
                              ps, pd)
    o1a, o1b = _sc1(h1a, h1b, s1t, d1t, src3, dst3, b1)
    return jnp.concatenate([o1a, o1b], axis=1)


# speedup vs baseline: 1.0265x; 1.0265x over previous
import jax
import jax.numpy as jnp
from jax import lax
from jax.experimental import pallas as pl
from jax.experimental.pallas import tpu as pltpu
from jax.experimental.pallas import tpu_sc as plsc

N = 10000
E = 160000
D = 256
HID = 256
NCLS = 32
NEG = 0.2

NTILES = 16
EPT = E // NTILES
BCH = 80
NCH = EPT // BCH
NROW = 80
NRCH = N // NROW
NRPT = -(-NRCH // NTILES)

f32 = jnp.float32
i32 = jnp.int32

BN = 2000
TCGRID = N // BN


def _tc1_body(x_ref, w0_ref, ssa_ref, sda_ref, ssb_ref, sdb_ref,
              ha_ref, hb_ref, asa_ref, ada_ref, asb_ref, adb_ref):
    h = jnp.dot(x_ref[...], w0_ref[...], preferred_element_type=f32)
    ha_ref[...] = h[:, :128]
    hb_ref[...] = h[:, 128:]
    asa_ref[...] = jnp.dot(h, ssa_ref[...], preferred_element_type=f32)
    ada_ref[...] = jnp.dot(h, sda_ref[...], preferred_element_type=f32)
    asb_ref[...] = jnp.dot(h, ssb_ref[...], preferred_element_type=f32)
    adb_ref[...] = jnp.dot(h, sdb_ref[...], preferred_element_type=f32)


def _tc1(x, w0, ssa, sda, ssb, sdb):
    full = lambda s: pl.BlockSpec(s, lambda i: (0, 0))
    row = lambda w: pl.BlockSpec((BN, w), lambda i: (i, 0))
    return pl.pallas_call(
        _tc1_body,
        grid=(TCGRID,),
        in_specs=[row(D), full((D, HID)), full((HID, 16)), full((HID, 16)),
                  full((HID, 16)), full((HID, 16))],
        out_specs=[row(128), row(128), row(16), row(16), row(16), row(16)],
        out_shape=[jax.ShapeDtypeStruct((N, 128), f32),
                   jax.ShapeDtypeStruct((N, 128), f32),
                   jax.ShapeDtypeStruct((N, 16), f32),
                   jax.ShapeDtypeStruct((N, 16), f32),
                   jax.ShapeDtypeStruct((N, 16), f32),
                   jax.ShapeDtypeStruct((N, 16), f32)],
    )(x, w0, ssa, sda, ssb, sdb)


def _tc2_body(ma_ref, mb_ref, da_ref, db_ref, b0a_ref, b0b_ref,
              w1a_ref, w1b_ref, ps_ref, pd_ref,
              h1a_ref, h1b_ref, s1_ref, d1_ref):
    eps = 1e-16

    def norm(m_ref, d_ref, b_ref):
        d = d_ref[...]
        den = jnp.concatenate(
            [jnp.broadcast_to(d[:, 0:1] + eps, (BN, 64)),
             jnp.broadcast_to(d[:, 1:2] + eps, (BN, 64))], axis=1)
        v = m_ref[...] / den + b_ref[...]
        return jnp.where(v > 0, v, jnp.exp(v) - 1.0)

    h0a = norm(ma_ref, da_ref, b0a_ref)
    h0b = norm(mb_ref, db_ref, b0b_ref)
    h1 = (jnp.dot(h0a, w1a_ref[...], preferred_element_type=f32) +
          jnp.dot(h0b, w1b_ref[...], preferred_element_type=f32))
    h1a_ref[...] = h1[:, :16]
    h1b_ref[...] = h1[:, 16:]
    s1_ref[...] = jnp.dot(h1, ps_ref[...], preferred_element_type=f32)
    d1_ref[...] = jnp.dot(h1, pd_ref[...], preferred_element_type=f32)


def _tc2(ma, mb, da, db, b0a, b0b, w1a, w1b, ps, pd):
    full = lambda s: pl.BlockSpec(s, lambda i: (0, 0))
    row = lambda w: pl.BlockSpec((BN, w), lambda i: (i, 0))
    return pl.pallas_call(
        _tc2_body,
        grid=(TCGRID,),
        in_specs=[row(128), row(128), row(16), row(16),
                  full((1, 128)), full((1, 128)),
                  full((128, NCLS)), full((128, NCLS)),
                  full((NCLS, 16)), full((NCLS, 16))],
        out_specs=[row(16), row(16), row(16), row(16)],
        out_shape=[jax.ShapeDtypeStruct((N, 16), f32),
                   jax.ShapeDtypeStruct((N, 16), f32),
                   jax.ShapeDtypeStruct((N, 16), f32),
                   jax.ShapeDtypeStruct((N, 16), f32)],
    )(ma, mb, da, db, b0a, b0b, w1a, w1b, ps, pd)


_GDN = lax.GatherDimensionNumbers(
    offset_dims=(), collapsed_slice_dims=(0,), start_index_map=(0,))


def _splat(v, j):
    idx = jnp.full((16, 1), j, dtype=i32)
    return lax.gather(v, idx, _GDN, (1,),
                      mode=lax.GatherScatterMode.PROMISE_IN_BOUNDS)


def _mesh():
    return plsc.VectorSubcoreMesh(core_axis_name="c", subcore_axis_name="s",
                                  num_cores=2, num_subcores=16)


def _cp():
    return pltpu.CompilerParams(use_tc_tiling_on_sc=False,
                                needs_layout_passes=False)


def _sc0_body(ha, hb, asa, ada, asb, adb, srch, dsth,
              msga, msgb, dena, denb,
              sidx2, didx2, av, dv, exv, rows, msg_sh, den_sh, gsem, ssem):
    c = lax.axis_index("c")
    s = lax.axis_index("s")
    lanelt2 = lax.broadcasted_iota(i32, (16,), 0) < 2
    zero16 = jnp.zeros((16,), f32)

    def zb(i, _):
        for k in range(8):
            rows[i, pl.ds(k * 16, 16)] = zero16
        exv[i, :] = zero16
        return 0
    lax.fori_loop(0, BCH, zb, 0)

    for k in range(NRPT):
        cid = s + NTILES * k

        def zc(lo=pl.multiple_of(cid * NROW, 8)):
            pltpu.sync_copy(rows, msg_sh.at[pl.ds(lo, NROW)])
            pltpu.sync_copy(exv, den_sh.at[pl.ds(lo, NROW)])
        pl.when(cid < NRCH)(zc)

    pltpu.sync_copy(srch.at[s], sidx2)
    pltpu.sync_copy(dsth.at[s], didx2)
    plsc.subcore_barrier()

    def run(h_t, as_t, ad_t):
        def chunk(j, _):
            si = sidx2.at[j]
            di = didx2.at[j]
            g1 = pltpu.async_copy(as_t.at[si], av, gsem)
            g2 = pltpu.async_copy(ad_t.at[di], dv, gsem)
            g3 = pltpu.async_copy(h_t.at[si], rows, gsem)
            g1.wait()
            g2.wait()
            g3.wait()

            def edge(i, _):
                e = av[i, :] + dv[i, :]
                e = jnp.where(e > 0, e, NEG * e)
                ex = jnp.where(lanelt2, jnp.exp(e), 0.0)
                exv[i, :] = ex
                s0 = _splat(ex, 0)
                s1 = _splat(ex, 1)
                for k in range(4):
                    rows[i, pl.ds(k * 16, 16)] = rows[i, pl.ds(k * 16, 16)] * s0
                for k in range(4, 8):
                    rows[i, pl.ds(k * 16, 16)] = rows[i, pl.ds(k * 16, 16)] * s1
                return 0
            lax.fori_loop(0, BCH, edge, 0, unroll=2)
            s1_ = pltpu.async_copy(exv, den_sh.at[di], ssem, add=True)
            s2_ = pltpu.async_copy(rows, msg_sh.at[di], ssem, add=True)
            s1_.wait()
            s2_.wait()
            return 0
        lax.fori_loop(0, NCH, chunk, 0)

    pl.when(c == 0)(lambda: run(ha, asa, ada))
    pl.when(c == 1)(lambda: run(hb, asb, adb))
    plsc.subcore_barrier()

    def writeout(msg_o, den_o):
        for k in range(NRPT):
            cid = s + NTILES * k

            def wc(lo=pl.multiple_of((s + NTILES * k) * NROW, 8)):
                pltpu.sync_copy(msg_sh.at[pl.ds(lo, NROW)], rows)
                pltpu.sync_copy(rows, msg_o.at[pl.ds(lo, NROW)])
                pltpu.sync_copy(den_sh.at[pl.ds(lo, NROW)], exv)
                pltpu.sync_copy(exv, den_o.at[pl.ds(lo, NROW)])
            pl.when(cid < NRCH)(wc)

    pl.when(c == 0)(lambda: writeout(msga, dena))
    pl.when(c == 1)(lambda: writeout(msgb, denb))


def _sc0(ha, hb, asa, ada, asb, adb, src3, dst3):
    fn = pl.kernel(
        _sc0_body,
        out_type=[jax.ShapeDtypeStruct((N, 128), f32),
                  jax.ShapeDtypeStruct((N, 128), f32),
                  jax.ShapeDtypeStruct((N, 16), f32),
                  jax.ShapeDtypeStruct((N, 16), f32)],
        mesh=_mesh(),
        compiler_params=_cp(),
        scratch_types=[pltpu.VMEM((NCH, BCH), i32),
                       pltpu.VMEM((NCH, BCH), i32),
                       pltpu.VMEM((BCH, 16), f32),
                       pltpu.VMEM((BCH, 16), f32),
                       pltpu.VMEM((BCH, 16), f32),
                       pltpu.VMEM((BCH, 128), f32),
                       pltpu.VMEM_SHARED((N, 128), f32),
                       pltpu.VMEM_SHARED((N, 16), f32),
                       pltpu.SemaphoreType.DMA,
                       pltpu.SemaphoreType.DMA],
    )
    return fn(ha, hb, asa, ada, asb, adb, src3, dst3)


def _sc1_body(h1a, h1b, s1t, d1t, srch, dsth, b1h,
              o1a, o1b,
              sidx2, didx2, av, dv, exv, rows, accb, denb_v, bvec,
              acc_sh, den_sh, gsem, ssem):
    c = lax.axis_index("c")
    s = lax.axis_index("s")
    lane0 = lax.broadcasted_iota(i32, (16,), 0) < 1
    zero16 = jnp.zeros((16,), f32)

    def zb(i, _):
        rows[i, :] = zero16
        exv[i, :] = zero16
        return 0
    lax.fori_loop(0, BCH, zb, 0)

    for k in range(NRPT):
        cid = s + NTILES * k

        def zc(lo=pl.multiple_of(cid * NROW, 8)):
            pltpu.sync_copy(rows, acc_sh.at[pl.ds(lo, NROW)])
            pltpu.sync_copy(exv, den_sh.at[pl.ds(lo, NROW)])
        pl.when(cid < NRCH)(zc)

    pltpu.sync_copy(srch.at[s], sidx2)
    pltpu.sync_copy(dsth.at[s], didx2)
    plsc.subcore_barrier()

    def run(h_t):
        def chunk(j, _):
            si = sidx2.at[j]
            di = didx2.at[j]
            g1 = pltpu.async_copy(s1t.at[si], av, gsem)
            g2 = pltpu.async_copy(d1t.at[di], dv, gsem)
            g3 = pltpu.async_copy(h_t.at[si], rows, gsem)
            g1.wait()
            g2.wait()
            g3.wait()

            def edge(i, _):
                e = av[i, :] + dv[i, :]
                e = jnp.where(e > 0, e, NEG * e)
                ex = jnp.where(lane0, jnp.exp(e), 0.0)
                exv[i, :] = ex
                rows[i, :] = rows[i, :] * _splat(ex, 0)
                return 0
            lax.fori_loop(0, BCH, edge, 0, unroll=2)
            s1_ = pltpu.async_copy(exv, den_sh.at[di], ssem, add=True)
            s2_ = pltpu.async_copy(rows, acc_sh.at[di], ssem, add=True)
            s1_.wait()
            s2_.wait()
            return 0
        lax.fori_loop(0, NCH, chunk, 0)

    pl.when(c == 0)(lambda: run(h1a))
    pl.when(c == 1)(lambda: run(h1b))
    plsc.subcore_barrier()

    pltpu.sync_copy(b1h.at[pl.ds(pl.multiple_of(c * 16, 16), 16)], bvec)
    bv = bvec[:]

    def writeout(o_ref):
        for k in range(NRPT):
            cid = s + NTILES * k

            def wc(lo=pl.multiple_of((s + NTILES * k) * NROW, 8)):
                pltpu.sync_copy(acc_sh.at[pl.ds(lo, NROW)], accb)
                pltpu.sync_copy(den_sh.at[pl.ds(lo, NROW)], denb_v)

                def node(i, _):
                    dn = _splat(denb_v[i, :], 0) + 1e-16
                    accb[i, :] = accb[i, :] / dn + bv
                    return 0
                lax.fori_loop(0, NROW, node, 0)
                pltpu.sync_copy(accb, o_ref.at[pl.ds(lo, NROW)])
            pl.when(cid < NRCH)(wc)

    pl.when(c == 0)(lambda: writeout(o1a))
    pl.when(c == 1)(lambda: writeout(o1b))


def _sc1(h1a, h1b, s1t, d1t, src3, dst3, b1):
    fn = pl.kernel(
        _sc1_body,
        out_type=[jax.ShapeDtypeStruct((N, 16), f32),
                  jax.ShapeDtypeStruct((N, 16), f32)],
        mesh=_mesh(),
        compiler_params=_cp(),
        scratch_types=[pltpu.VMEM((NCH, BCH), i32),
                       pltpu.VMEM((NCH, BCH), i32),
                       pltpu.VMEM((BCH, 16), f32),
                       pltpu.VMEM((BCH, 16), f32),
                       pltpu.VMEM((BCH, 16), f32),
                       pltpu.VMEM((BCH, 16), f32),
                       pltpu.VMEM((NROW, 16), f32),
                       pltpu.VMEM((NROW, 16), f32),
                       pltpu.VMEM((16,), f32),
                       pltpu.VMEM_SHARED((N, 16), f32),
                       pltpu.VMEM_SHARED((N, 16), f32),
                       pltpu.SemaphoreType.DMA,
                       pltpu.SemaphoreType.DMA],
    )
    return fn(h1a, h1b, s1t, d1t, src3, dst3, b1)


def _head_sel(att, heads):
    sel = jnp.zeros((HID, 16), f32)
    for j, h in enumerate(heads):
        sel = sel.at[h * 64:(h + 1) * 64, j].set(att[h])
    return sel


def kernel(x, edge_index, W0, att_src0, att_dst0, b0, W1, att_src1,
           att_dst1, b1):
    src3 = edge_index[0].astype(i32).reshape(NTILES, NCH, BCH)
    dst3 = edge_index[1].astype(i32).reshape(NTILES, NCH, BCH)

    ssa = _head_sel(att_src0, (0, 1))
    sda = _head_sel(att_dst0, (0, 1))
    ssb = _head_sel(att_src0, (2, 3))
    sdb = _head_sel(att_dst0, (2, 3))

    ha, hb, asa, ada, asb, adb = _tc1(x, W0, ssa, sda, ssb, sdb)
    msga, msgb, dena, denb = _sc0(ha, hb, asa, ada, asb, adb, src3, dst3)

    b0a = b0[:128].reshape(1, 128)
    b0b = b0[128:].reshape(1, 128)
    w1a = W1[:128]
    w1b = W1[128:]
    ps = jnp.zeros((NCLS, 16), f32).at[:, 0].set(att_src1[0])
    pd = jnp.zeros((NCLS, 16), f32).at[:, 0].set(att_dst1[0])

    h1a, h1b, s1t, d1t = _tc2(msga, msgb, dena, denb, b0a, b0b, w1a, w1b,
                              ps, pd)
    o1a, o1b = _sc1(h1a, h1b, s1t, d1t, src3, dst3, b1)
    return jnp.concatenate([o1a, o1b], axis=1)
